# Initial kernel scaffold; baseline (speedup 1.0000x reference)
import jax
import jax.numpy as jnp
from jax.experimental import pallas as pl


def kernel(position_ids, hidden_states, norm1_w, norm1_b, norm2_w, norm2_b, Wqkv, Wout, Wrouter, ws, w2s):
    raise NotImplementedError("write your pallas kernel here")


# speedup vs baseline: 1.3569x; 1.3569x over previous
import functools

import jax
import jax.numpy as jnp
from jax.experimental import pallas as pl

T = 2048
D = 768
H = 12
KVH = 4
HD = 64
E = 8
K = 2
I = 1536
THETA = 10000.0
CLIP = 8.0
QW = H * HD
KVW = KVH * HD
HALF = HD // 2
SCALE = HD ** -0.5

BT1 = 256
BTQ = 512
BTM = 512


def _silu(x):
    return x / (1.0 + jnp.exp(-x))


def _k1_body(hs_ref, w_ref, b_ref, wqkv_ref, cos_ref, sin_ref,
             q_ref, k_ref, v_ref):
    x = hs_ref[...]
    mu = jnp.mean(x, axis=-1, keepdims=True)
    var = jnp.mean((x - mu) ** 2, axis=-1, keepdims=True)
    xn = (x - mu) * jax.lax.rsqrt(var + 1e-5) * w_ref[...] + b_ref[...]
    qkv = jax.lax.dot_general(
        xn.astype(jnp.bfloat16), wqkv_ref[...].astype(jnp.bfloat16),
        (((1,), (1,)), ((), ())), preferred_element_type=jnp.float32)
    qkv = jnp.clip(qkv, -CLIP, CLIP)
    cos = cos_ref[...]
    sin = sin_ref[...]
    for h in range(H):
        base = h * HD
        x1 = qkv[:, base:base + HALF]
        x2 = qkv[:, base + HALF:base + HD]
        q_ref[:, base:base + HALF] = (x1 * cos - x2 * sin).astype(jnp.bfloat16)
        q_ref[:, base + HALF:base + HD] = (x1 * sin + x2 * cos).astype(jnp.bfloat16)
    for h in range(KVH):
        base = QW + h * HD
        ob = h * HD
        x1 = qkv[:, base:base + HALF]
        x2 = qkv[:, base + HALF:base + HD]
        k_ref[:, ob:ob + HALF] = (x1 * cos - x2 * sin).astype(jnp.bfloat16)
        k_ref[:, ob + HALF:ob + HD] = (x1 * sin + x2 * cos).astype(jnp.bfloat16)
    v_ref[...] = qkv[:, QW + KVW:].astype(jnp.bfloat16)


def _run_k1(hs, n1w, n1b, wqkv, cos, sin):
    grid = (T // BT1,)
    return pl.pallas_call(
        _k1_body,
        grid=grid,
        in_specs=[
            pl.BlockSpec((BT1, D), lambda i: (i, 0)),
            pl.BlockSpec((D,), lambda i: (0,)),
            pl.BlockSpec((D,), lambda i: (0,)),
            pl.BlockSpec((QW + 2 * KVW, D), lambda i: (0, 0)),
            pl.BlockSpec((BT1, HALF), lambda i: (i, 0)),
            pl.BlockSpec((BT1, HALF), lambda i: (i, 0)),
        ],
        out_specs=[
            pl.BlockSpec((BT1, QW), lambda i: (i, 0)),
            pl.BlockSpec((BT1, KVW), lambda i: (i, 0)),
            pl.BlockSpec((BT1, KVW), lambda i: (i, 0)),
        ],
        out_shape=[
            jax.ShapeDtypeStruct((T, QW), jnp.bfloat16),
            jax.ShapeDtypeStruct((T, KVW), jnp.bfloat16),
            jax.ShapeDtypeStruct((T, KVW), jnp.bfloat16),
        ],
    )(hs, n1w, n1b, wqkv, cos, sin)


def _k2_body(q_ref, k_ref, v_ref, o_ref):
    qi = pl.program_id(1)
    q = q_ref[0]
    k = k_ref[0]
    s = jax.lax.dot_general(q, k, (((1,), (1,)), ((), ())),
                            preferred_element_type=jnp.float32) * SCALE
    rows = jax.lax.broadcasted_iota(jnp.int32, (BTQ, T), 0) + qi * BTQ
    cols = jax.lax.broadcasted_iota(jnp.int32, (BTQ, T), 1)
    s = jnp.where(rows >= cols, s, -1e30)
    m = jnp.max(s, axis=-1, keepdims=True)
    p = jnp.exp(s - m)
    l = jnp.sum(p, axis=-1, keepdims=True)
    p = (p / l).astype(jnp.bfloat16)
    o_ref[0] = jnp.dot(p, v_ref[0], preferred_element_type=jnp.float32)


def _run_k2(q, k, v):
    rep = H // KVH
    grid = (H, T // BTQ)
    return pl.pallas_call(
        _k2_body,
        grid=grid,
        in_specs=[
            pl.BlockSpec((1, BTQ, HD), lambda h, i: (h, i, 0)),
            pl.BlockSpec((1, T, HD), lambda h, i: (h // rep, 0, 0)),
            pl.BlockSpec((1, T, HD), lambda h, i: (h // rep, 0, 0)),
        ],
        out_specs=pl.BlockSpec((1, BTQ, HD), lambda h, i: (h, i, 0)),
        out_shape=jax.ShapeDtypeStruct((H, T, HD), jnp.float32),
    )(q, k, v)


def _k3_body(attn_ref, wout_ref, res_ref, w_ref, b_ref, wr_ref,
             h_ref, x2_ref, comb_ref):
    a = attn_ref[...].astype(jnp.bfloat16)
    h = res_ref[...] + jax.lax.dot_general(
        a, wout_ref[...].astype(jnp.bfloat16), (((1,), (1,)), ((), ())),
        preferred_element_type=jnp.float32)
    h_ref[...] = h
    mu = jnp.mean(h, axis=-1, keepdims=True)
    var = jnp.mean((h - mu) ** 2, axis=-1, keepdims=True)
    x2 = (h - mu) * jax.lax.rsqrt(var + 1e-5) * w_ref[...] + b_ref[...]
    x2_ref[...] = x2
    logits = jax.lax.dot_general(x2, wr_ref[...], (((1,), (1,)), ((), ())),
                                 preferred_element_type=jnp.float32)
    mx = jnp.max(logits, axis=-1, keepdims=True)
    ex = jnp.exp(logits - mx)
    w_all = ex / jnp.sum(ex, axis=-1, keepdims=True)
    idx = jax.lax.broadcasted_iota(jnp.int32, (BT1, E), 1)
    m1 = jnp.max(w_all, axis=-1, keepdims=True)
    am1 = jnp.min(jnp.where(w_all == m1, idx, E), axis=-1, keepdims=True)
    is1 = idx == am1
    w_rest = jnp.where(is1, -1.0, w_all)
    m2 = jnp.max(w_rest, axis=-1, keepdims=True)
    am2 = jnp.min(jnp.where(w_rest == m2, idx, E), axis=-1, keepdims=True)
    tot = m1 + m2
    comb_ref[...] = (jnp.where(is1, m1, 0.0)
                     + jnp.where(idx == am2, m2, 0.0)) / tot


def _run_k3(attn, wout, res, n2w, n2b, wr):
    grid = (T // BT1,)
    return pl.pallas_call(
        _k3_body,
        grid=grid,
        in_specs=[
            pl.BlockSpec((BT1, QW), lambda i: (i, 0)),
            pl.BlockSpec((D, QW), lambda i: (0, 0)),
            pl.BlockSpec((BT1, D), lambda i: (i, 0)),
            pl.BlockSpec((D,), lambda i: (0,)),
            pl.BlockSpec((D,), lambda i: (0,)),
            pl.BlockSpec((E, D), lambda i: (0, 0)),
        ],
        out_specs=[
            pl.BlockSpec((BT1, D), lambda i: (i, 0)),
            pl.BlockSpec((BT1, D), lambda i: (i, 0)),
            pl.BlockSpec((BT1, E), lambda i: (i, 0)),
        ],
        out_shape=[
            jax.ShapeDtypeStruct((T, D), jnp.float32),
            jax.ShapeDtypeStruct((T, D), jnp.float32),
            jax.ShapeDtypeStruct((T, E), jnp.float32),
        ],
    )(attn, wout, res, n2w, n2b, wr)


def _k4_body(x2_ref, comb_ref, res_ref, ws_ref, w2_ref, out_ref):
    e = pl.program_id(1)
    x = x2_ref[...].astype(jnp.bfloat16)
    w1 = ws_ref[0, :I, :].astype(jnp.bfloat16)
    v1 = ws_ref[0, I:, :].astype(jnp.bfloat16)
    g = jax.lax.dot_general(x, w1, (((1,), (1,)), ((), ())),
                            preferred_element_type=jnp.float32)
    u = jax.lax.dot_general(x, v1, (((1,), (1,)), ((), ())),
                            preferred_element_type=jnp.float32)
    act = (_silu(g) * u).astype(jnp.bfloat16)
    y = jax.lax.dot_general(act, w2_ref[0].astype(jnp.bfloat16),
                            (((1,), (1,)), ((), ())),
                            preferred_element_type=jnp.float32)
    eid = jax.lax.broadcasted_iota(jnp.int32, (E, 1), 0)
    onehot = (eid == e).astype(jnp.float32)
    wcol = jnp.dot(comb_ref[...], onehot, preferred_element_type=jnp.float32)

    @pl.when(e == 0)
    def _():
        out_ref[...] = res_ref[...] + wcol * y

    @pl.when(e > 0)
    def _():
        out_ref[...] += wcol * y


def _run_k4(x2, comb, res, ws, w2s):
    grid = (T // BTM, E)
    return pl.pallas_call(
        _k4_body,
        grid=grid,
        in_specs=[
            pl.BlockSpec((BTM, D), lambda t, e: (t, 0)),
            pl.BlockSpec((BTM, E), lambda t, e: (t, 0)),
            pl.BlockSpec((BTM, D), lambda t, e: (t, 0)),
            pl.BlockSpec((1, 2 * I, D), lambda t, e: (e, 0, 0)),
            pl.BlockSpec((1, D, I), lambda t, e: (e, 0, 0)),
        ],
        out_specs=pl.BlockSpec((BTM, D), lambda t, e: (t, 0)),
        out_shape=jax.ShapeDtypeStruct((T, D), jnp.float32),
    )(x2, comb, res, ws, w2s)


def kernel(position_ids, hidden_states, norm1_w, norm1_b, norm2_w, norm2_b,
           Wqkv, Wout, Wrouter, ws, w2s):
    inv = 1.0 / (THETA ** (jnp.arange(HALF, dtype=jnp.float32) / HALF))
    ang = position_ids.astype(jnp.float32)[:, None] * inv[None, :]
    cos = jnp.cos(ang)
    sin = jnp.sin(ang)

    q, k, v = _run_k1(hidden_states, norm1_w, norm1_b, Wqkv, cos, sin)
    qh = q.reshape(T, H, HD).transpose(1, 0, 2)
    kh = k.reshape(T, KVH, HD).transpose(1, 0, 2)
    vh = v.reshape(T, KVH, HD).transpose(1, 0, 2)
    attn = _run_k2(qh, kh, vh).transpose(1, 0, 2).reshape(T, QW)
    h, x2, comb = _run_k3(attn, Wout, hidden_states, norm2_w, norm2_b, Wrouter)
    return _run_k4(x2, comb, h, ws, w2s)


# speedup vs baseline: 1.8005x; 1.3269x over previous
import functools

import jax
import jax.numpy as jnp
from jax.experimental import pallas as pl

T = 2048
D = 768
H = 12
KVH = 4
HD = 64
E = 8
K = 2
I = 1536
THETA = 10000.0
CLIP = 8.0
QW = H * HD
KVW = KVH * HD
HALF = HD // 2
SCALE = HD ** -0.5
REP = H // KVH

BT1 = 256
BTQ = 512
BK = 512
BTM = 512


def _silu(x):
    return x / (1.0 + jnp.exp(-x))


def _k1_body(hs_ref, w_ref, b_ref, wqkv_ref, cos_ref, sin_ref,
             q_ref, k_ref, v_ref):
    x = hs_ref[...]
    mu = jnp.mean(x, axis=-1, keepdims=True)
    var = jnp.mean((x - mu) ** 2, axis=-1, keepdims=True)
    xn = (x - mu) * jax.lax.rsqrt(var + 1e-5) * w_ref[...] + b_ref[...]
    qkv = jax.lax.dot_general(
        xn.astype(jnp.bfloat16), wqkv_ref[...].astype(jnp.bfloat16),
        (((1,), (1,)), ((), ())), preferred_element_type=jnp.float32)
    qkv = jnp.clip(qkv, -CLIP, CLIP)
    cos = cos_ref[...]
    sin = sin_ref[...]
    for h in range(H):
        base = h * HD
        x1 = qkv[:, base:base + HALF]
        x2 = qkv[:, base + HALF:base + HD]
        q_ref[h, :, :HALF] = (x1 * cos - x2 * sin).astype(jnp.bfloat16)
        q_ref[h, :, HALF:] = (x1 * sin + x2 * cos).astype(jnp.bfloat16)
    for h in range(KVH):
        base = QW + h * HD
        x1 = qkv[:, base:base + HALF]
        x2 = qkv[:, base + HALF:base + HD]
        k_ref[h, :, :HALF] = (x1 * cos - x2 * sin).astype(jnp.bfloat16)
        k_ref[h, :, HALF:] = (x1 * sin + x2 * cos).astype(jnp.bfloat16)
        vbase = QW + KVW + h * HD
        v_ref[h, :, :] = qkv[:, vbase:vbase + HD].astype(jnp.bfloat16)


def _run_k1(hs, n1w, n1b, wqkv, cos, sin):
    grid = (T // BT1,)
    return pl.pallas_call(
        _k1_body,
        grid=grid,
        in_specs=[
            pl.BlockSpec((BT1, D), lambda i: (i, 0)),
            pl.BlockSpec((D,), lambda i: (0,)),
            pl.BlockSpec((D,), lambda i: (0,)),
            pl.BlockSpec((QW + 2 * KVW, D), lambda i: (0, 0)),
            pl.BlockSpec((BT1, HALF), lambda i: (i, 0)),
            pl.BlockSpec((BT1, HALF), lambda i: (i, 0)),
        ],
        out_specs=[
            pl.BlockSpec((H, BT1, HD), lambda i: (0, i, 0)),
            pl.BlockSpec((KVH, BT1, HD), lambda i: (0, i, 0)),
            pl.BlockSpec((KVH, BT1, HD), lambda i: (0, i, 0)),
        ],
        out_shape=[
            jax.ShapeDtypeStruct((H, T, HD), jnp.bfloat16),
            jax.ShapeDtypeStruct((KVH, T, HD), jnp.bfloat16),
            jax.ShapeDtypeStruct((KVH, T, HD), jnp.bfloat16),
        ],
    )(hs, n1w, n1b, wqkv, cos, sin)


def _k2_body(q_ref, k_ref, v_ref, o_ref):
    qi = pl.program_id(1)
    q = q_ref[0]

    def body(j, carry):
        m, l, acc = carry
        kc = k_ref[0, pl.ds(j * BK, BK), :]
        s = jax.lax.dot_general(q, kc, (((1,), (1,)), ((), ())),
                                preferred_element_type=jnp.float32) * SCALE
        rows = jax.lax.broadcasted_iota(jnp.int32, (BTQ, BK), 0) + qi * BTQ
        cols = jax.lax.broadcasted_iota(jnp.int32, (BTQ, BK), 1) + j * BK
        s = jnp.where(rows >= cols, s, -1e30)
        mc = jnp.max(s, axis=-1, keepdims=True)
        mn = jnp.maximum(m, mc)
        p = jnp.exp(s - mn)
        corr = jnp.exp(m - mn)
        l = l * corr + jnp.sum(p, axis=-1, keepdims=True)
        vc = v_ref[0, pl.ds(j * BK, BK), :]
        acc = acc * corr + jnp.dot(p.astype(jnp.bfloat16), vc,
                                   preferred_element_type=jnp.float32)
        return mn, l, acc

    m0 = jnp.full((BTQ, 1), -1e30, jnp.float32)
    l0 = jnp.zeros((BTQ, 1), jnp.float32)
    a0 = jnp.zeros((BTQ, HD), jnp.float32)
    m, l, acc = jax.lax.fori_loop(0, qi + 1, body, (m0, l0, a0))
    o_ref[0] = acc / l


def _run_k2(q, k, v):
    grid = (H, T // BTQ)
    return pl.pallas_call(
        _k2_body,
        grid=grid,
        in_specs=[
            pl.BlockSpec((1, BTQ, HD), lambda h, i: (h, i, 0)),
            pl.BlockSpec((1, T, HD), lambda h, i: (h // REP, 0, 0)),
            pl.BlockSpec((1, T, HD), lambda h, i: (h // REP, 0, 0)),
        ],
        out_specs=pl.BlockSpec((1, BTQ, HD), lambda h, i: (h, i, 0)),
        out_shape=jax.ShapeDtypeStruct((H, T, HD), jnp.float32),
    )(q, k, v)


def _k3_body(attn_ref, wout_ref, res_ref, w_ref, b_ref, wr_ref,
             h_ref, x2_ref, comb_ref):
    a = jnp.concatenate([attn_ref[h] for h in range(H)], axis=-1)
    a = a.astype(jnp.bfloat16)
    h = res_ref[...] + jax.lax.dot_general(
        a, wout_ref[...].astype(jnp.bfloat16), (((1,), (1,)), ((), ())),
        preferred_element_type=jnp.float32)
    h_ref[...] = h
    mu = jnp.mean(h, axis=-1, keepdims=True)
    var = jnp.mean((h - mu) ** 2, axis=-1, keepdims=True)
    x2 = (h - mu) * jax.lax.rsqrt(var + 1e-5) * w_ref[...] + b_ref[...]
    x2_ref[...] = x2
    logits = jax.lax.dot_general(x2, wr_ref[...], (((1,), (1,)), ((), ())),
                                 preferred_element_type=jnp.float32)
    mx = jnp.max(logits, axis=-1, keepdims=True)
    ex = jnp.exp(logits - mx)
    w_all = ex / jnp.sum(ex, axis=-1, keepdims=True)
    idx = jax.lax.broadcasted_iota(jnp.int32, (BT1, E), 1)
    m1 = jnp.max(w_all, axis=-1, keepdims=True)
    am1 = jnp.min(jnp.where(w_all == m1, idx, E), axis=-1, keepdims=True)
    is1 = idx == am1
    w_rest = jnp.where(is1, -1.0, w_all)
    m2 = jnp.max(w_rest, axis=-1, keepdims=True)
    am2 = jnp.min(jnp.where(w_rest == m2, idx, E), axis=-1, keepdims=True)
    tot = m1 + m2
    comb_ref[...] = (jnp.where(is1, m1, 0.0)
                     + jnp.where(idx == am2, m2, 0.0)) / tot


def _run_k3(attn, wout, res, n2w, n2b, wr):
    grid = (T // BT1,)
    return pl.pallas_call(
        _k3_body,
        grid=grid,
        in_specs=[
            pl.BlockSpec((H, BT1, HD), lambda i: (0, i, 0)),
            pl.BlockSpec((D, QW), lambda i: (0, 0)),
            pl.BlockSpec((BT1, D), lambda i: (i, 0)),
            pl.BlockSpec((D,), lambda i: (0,)),
            pl.BlockSpec((D,), lambda i: (0,)),
            pl.BlockSpec((E, D), lambda i: (0, 0)),
        ],
        out_specs=[
            pl.BlockSpec((BT1, D), lambda i: (i, 0)),
            pl.BlockSpec((BT1, D), lambda i: (i, 0)),
            pl.BlockSpec((BT1, E), lambda i: (i, 0)),
        ],
        out_shape=[
            jax.ShapeDtypeStruct((T, D), jnp.float32),
            jax.ShapeDtypeStruct((T, D), jnp.float32),
            jax.ShapeDtypeStruct((T, E), jnp.float32),
        ],
    )(attn, wout, res, n2w, n2b, wr)


def _k4_body(x2_ref, comb_ref, res_ref, ws_ref, w2_ref, out_ref):
    e = pl.program_id(0)
    w1 = ws_ref[0, :I, :].astype(jnp.bfloat16)
    v1 = ws_ref[0, I:, :].astype(jnp.bfloat16)
    w2 = w2_ref[0].astype(jnp.bfloat16)
    eid = jax.lax.broadcasted_iota(jnp.int32, (E, 1), 0)
    onehot = (eid == e).astype(jnp.float32)
    for tc in range(T // BTM):
        sl = pl.ds(tc * BTM, BTM)
        x = x2_ref[sl, :].astype(jnp.bfloat16)
        g = jax.lax.dot_general(x, w1, (((1,), (1,)), ((), ())),
                                preferred_element_type=jnp.float32)
        u = jax.lax.dot_general(x, v1, (((1,), (1,)), ((), ())),
                                preferred_element_type=jnp.float32)
        act = (_silu(g) * u).astype(jnp.bfloat16)
        y = jax.lax.dot_general(act, w2, (((1,), (1,)), ((), ())),
                                preferred_element_type=jnp.float32)
        wcol = jnp.dot(comb_ref[sl, :], onehot,
                       preferred_element_type=jnp.float32)

        @pl.when(e == 0)
        def _():
            out_ref[sl, :] = res_ref[sl, :] + wcol * y

        @pl.when(e > 0)
        def _():
            out_ref[sl, :] += wcol * y


def _run_k4(x2, comb, res, ws, w2s):
    grid = (E,)
    return pl.pallas_call(
        _k4_body,
        grid=grid,
        in_specs=[
            pl.BlockSpec((T, D), lambda e: (0, 0)),
            pl.BlockSpec((T, E), lambda e: (0, 0)),
            pl.BlockSpec((T, D), lambda e: (0, 0)),
            pl.BlockSpec((1, 2 * I, D), lambda e: (e, 0, 0)),
            pl.BlockSpec((1, D, I), lambda e: (e, 0, 0)),
        ],
        out_specs=pl.BlockSpec((T, D), lambda e: (0, 0)),
        out_shape=jax.ShapeDtypeStruct((T, D), jnp.float32),
    )(x2, comb, res, ws, w2s)


def kernel(position_ids, hidden_states, norm1_w, norm1_b, norm2_w, norm2_b,
           Wqkv, Wout, Wrouter, ws, w2s):
    inv = 1.0 / (THETA ** (jnp.arange(HALF, dtype=jnp.float32) / HALF))
    ang = position_ids.astype(jnp.float32)[:, None] * inv[None, :]
    cos = jnp.cos(ang)
    sin = jnp.sin(ang)

    q, k, v = _run_k1(hidden_states, norm1_w, norm1_b, Wqkv, cos, sin)
    attn = _run_k2(q, k, v)
    h, x2, comb = _run_k3(attn, Wout, hidden_states, norm2_w, norm2_b, Wrouter)
    return _run_k4(x2, comb, h, ws, w2s)


# speedup vs baseline: 1.8529x; 1.0291x over previous
import functools

import jax
import jax.numpy as jnp
from jax.experimental import pallas as pl

T = 2048
D = 768
H = 12
KVH = 4
HD = 64
E = 8
K = 2
I = 1536
THETA = 10000.0
CLIP = 8.0
QW = H * HD
KVW = KVH * HD
HALF = HD // 2
SCALE = HD ** -0.5
REP = H // KVH

BT1 = 256
BTQ = 512
BK = 512
BTM = 512


def _silu(x):
    return x / (1.0 + jnp.exp(-x))


def _k1_body(hs_ref, w_ref, b_ref, wqkv_ref, cos_ref, sin_ref,
             q_ref, k_ref, v_ref):
    x = hs_ref[...]
    mu = jnp.mean(x, axis=-1, keepdims=True)
    var = jnp.mean((x - mu) ** 2, axis=-1, keepdims=True)
    xn = (x - mu) * jax.lax.rsqrt(var + 1e-5) * w_ref[...] + b_ref[...]
    qkv = jax.lax.dot_general(
        xn.astype(jnp.bfloat16), wqkv_ref[...].astype(jnp.bfloat16),
        (((1,), (1,)), ((), ())), preferred_element_type=jnp.float32)
    qkv = jnp.clip(qkv, -CLIP, CLIP)
    cos = cos_ref[...]
    sin = sin_ref[...]
    for h in range(H):
        base = h * HD
        x1 = qkv[:, base:base + HALF]
        x2 = qkv[:, base + HALF:base + HD]
        q_ref[h, :, :HALF] = ((x1 * cos - x2 * sin) * SCALE).astype(jnp.bfloat16)
        q_ref[h, :, HALF:] = ((x1 * sin + x2 * cos) * SCALE).astype(jnp.bfloat16)
    for h in range(KVH):
        base = QW + h * HD
        x1 = qkv[:, base:base + HALF]
        x2 = qkv[:, base + HALF:base + HD]
        k_ref[h, :, :HALF] = (x1 * cos - x2 * sin).astype(jnp.bfloat16)
        k_ref[h, :, HALF:] = (x1 * sin + x2 * cos).astype(jnp.bfloat16)
        vbase = QW + KVW + h * HD
        v_ref[h, :, :] = qkv[:, vbase:vbase + HD].astype(jnp.bfloat16)


def _run_k1(hs, n1w, n1b, wqkv, cos, sin):
    grid = (T // BT1,)
    return pl.pallas_call(
        _k1_body,
        grid=grid,
        in_specs=[
            pl.BlockSpec((BT1, D), lambda i: (i, 0)),
            pl.BlockSpec((D,), lambda i: (0,)),
            pl.BlockSpec((D,), lambda i: (0,)),
            pl.BlockSpec((QW + 2 * KVW, D), lambda i: (0, 0)),
            pl.BlockSpec((BT1, HALF), lambda i: (i, 0)),
            pl.BlockSpec((BT1, HALF), lambda i: (i, 0)),
        ],
        out_specs=[
            pl.BlockSpec((H, BT1, HD), lambda i: (0, i, 0)),
            pl.BlockSpec((KVH, BT1, HD), lambda i: (0, i, 0)),
            pl.BlockSpec((KVH, BT1, HD), lambda i: (0, i, 0)),
        ],
        out_shape=[
            jax.ShapeDtypeStruct((H, T, HD), jnp.bfloat16),
            jax.ShapeDtypeStruct((KVH, T, HD), jnp.bfloat16),
            jax.ShapeDtypeStruct((KVH, T, HD), jnp.bfloat16),
        ],
    )(hs, n1w, n1b, wqkv, cos, sin)


def _k2_body(q_ref, k_ref, v_ref, o_ref):
    qi = pl.program_id(1)
    q = q_ref[0]

    def step(j, carry, masked):
        m, l, acc = carry
        kc = k_ref[0, pl.ds(j * BK, BK), :]
        s = jax.lax.dot_general(q, kc, (((1,), (1,)), ((), ())),
                                preferred_element_type=jnp.float32)
        if masked:
            rows = jax.lax.broadcasted_iota(jnp.int32, (BTQ, BK), 0)
            cols = jax.lax.broadcasted_iota(jnp.int32, (BTQ, BK), 1)
            s = jnp.where(rows >= cols, s, -1e30)
        mc = jnp.max(s, axis=-1, keepdims=True)
        mn = jnp.maximum(m, mc)
        p = jnp.exp(s - mn)
        corr = jnp.exp(m - mn)
        l = l * corr + jnp.sum(p, axis=-1, keepdims=True)
        vc = v_ref[0, pl.ds(j * BK, BK), :]
        acc = acc * corr + jnp.dot(p.astype(jnp.bfloat16), vc,
                                   preferred_element_type=jnp.float32)
        return mn, l, acc

    m0 = jnp.full((BTQ, 1), -1e30, jnp.float32)
    l0 = jnp.zeros((BTQ, 1), jnp.float32)
    a0 = jnp.zeros((BTQ, HD), jnp.float32)
    carry = jax.lax.fori_loop(0, qi, lambda j, c: step(j, c, False),
                              (m0, l0, a0))
    m, l, acc = step(qi, carry, True)
    o_ref[0] = acc / l


def _run_k2(q, k, v):
    grid = (H, T // BTQ)
    return pl.pallas_call(
        _k2_body,
        grid=grid,
        in_specs=[
            pl.BlockSpec((1, BTQ, HD), lambda h, i: (h, i, 0)),
            pl.BlockSpec((1, T, HD), lambda h, i: (h // REP, 0, 0)),
            pl.BlockSpec((1, T, HD), lambda h, i: (h // REP, 0, 0)),
        ],
        out_specs=pl.BlockSpec((1, BTQ, HD), lambda h, i: (h, i, 0)),
        out_shape=jax.ShapeDtypeStruct((H, T, HD), jnp.float32),
    )(q, k, v)


def _k3_body(attn_ref, wout_ref, res_ref, w_ref, b_ref, wr_ref,
             h_ref, x2_ref, comb_ref):
    a = jnp.concatenate([attn_ref[h] for h in range(H)], axis=-1)
    a = a.astype(jnp.bfloat16)
    h = res_ref[...] + jax.lax.dot_general(
        a, wout_ref[...].astype(jnp.bfloat16), (((1,), (1,)), ((), ())),
        preferred_element_type=jnp.float32)
    h_ref[...] = h
    mu = jnp.mean(h, axis=-1, keepdims=True)
    var = jnp.mean((h - mu) ** 2, axis=-1, keepdims=True)
    x2 = (h - mu) * jax.lax.rsqrt(var + 1e-5) * w_ref[...] + b_ref[...]
    x2_ref[...] = x2
    logits = jax.lax.dot_general(x2, wr_ref[...], (((1,), (1,)), ((), ())),
                                 preferred_element_type=jnp.float32)
    mx = jnp.max(logits, axis=-1, keepdims=True)
    ex = jnp.exp(logits - mx)
    w_all = ex / jnp.sum(ex, axis=-1, keepdims=True)
    idx = jax.lax.broadcasted_iota(jnp.int32, (BT1, E), 1)
    m1 = jnp.max(w_all, axis=-1, keepdims=True)
    am1 = jnp.min(jnp.where(w_all == m1, idx, E), axis=-1, keepdims=True)
    is1 = idx == am1
    w_rest = jnp.where(is1, -1.0, w_all)
    m2 = jnp.max(w_rest, axis=-1, keepdims=True)
    am2 = jnp.min(jnp.where(w_rest == m2, idx, E), axis=-1, keepdims=True)
    tot = m1 + m2
    comb_ref[...] = (jnp.where(is1, m1, 0.0)
                     + jnp.where(idx == am2, m2, 0.0)) / tot


def _run_k3(attn, wout, res, n2w, n2b, wr):
    grid = (T // BT1,)
    return pl.pallas_call(
        _k3_body,
        grid=grid,
        in_specs=[
            pl.BlockSpec((H, BT1, HD), lambda i: (0, i, 0)),
            pl.BlockSpec((D, QW), lambda i: (0, 0)),
            pl.BlockSpec((BT1, D), lambda i: (i, 0)),
            pl.BlockSpec((D,), lambda i: (0,)),
            pl.BlockSpec((D,), lambda i: (0,)),
            pl.BlockSpec((E, D), lambda i: (0, 0)),
        ],
        out_specs=[
            pl.BlockSpec((BT1, D), lambda i: (i, 0)),
            pl.BlockSpec((BT1, D), lambda i: (i, 0)),
            pl.BlockSpec((BT1, E), lambda i: (i, 0)),
        ],
        out_shape=[
            jax.ShapeDtypeStruct((T, D), jnp.float32),
            jax.ShapeDtypeStruct((T, D), jnp.float32),
            jax.ShapeDtypeStruct((T, E), jnp.float32),
        ],
    )(attn, wout, res, n2w, n2b, wr)


def _k4_body(x2_ref, comb_ref, res_ref, ws_ref, w2_ref, out_ref):
    e = pl.program_id(0)
    w1 = ws_ref[0, :I, :].astype(jnp.bfloat16)
    v1 = ws_ref[0, I:, :].astype(jnp.bfloat16)
    w2 = w2_ref[0].astype(jnp.bfloat16)
    eid = jax.lax.broadcasted_iota(jnp.int32, (E, 1), 0)
    onehot = (eid == e).astype(jnp.float32)
    for tc in range(T // BTM):
        sl = pl.ds(tc * BTM, BTM)
        x = x2_ref[sl, :].astype(jnp.bfloat16)
        g = jax.lax.dot_general(x, w1, (((1,), (1,)), ((), ())),
                                preferred_element_type=jnp.float32)
        u = jax.lax.dot_general(x, v1, (((1,), (1,)), ((), ())),
                                preferred_element_type=jnp.float32)
        act = (_silu(g) * u).astype(jnp.bfloat16)
        y = jax.lax.dot_general(act, w2, (((1,), (1,)), ((), ())),
                                preferred_element_type=jnp.float32)
        wcol = jnp.dot(comb_ref[sl, :], onehot,
                       preferred_element_type=jnp.float32)

        @pl.when(e == 0)
        def _():
            out_ref[sl, :] = res_ref[sl, :] + wcol * y

        @pl.when(e > 0)
        def _():
            out_ref[sl, :] += wcol * y


def _run_k4(x2, comb, res, ws, w2s):
    grid = (E,)
    return pl.pallas_call(
        _k4_body,
        grid=grid,
        in_specs=[
            pl.BlockSpec((T, D), lambda e: (0, 0)),
            pl.BlockSpec((T, E), lambda e: (0, 0)),
            pl.BlockSpec((T, D), lambda e: (0, 0)),
            pl.BlockSpec((1, 2 * I, D), lambda e: (e, 0, 0)),
            pl.BlockSpec((1, D, I), lambda e: (e, 0, 0)),
        ],
        out_specs=pl.BlockSpec((T, D), lambda e: (0, 0)),
        out_shape=jax.ShapeDtypeStruct((T, D), jnp.float32),
    )(x2, comb, res, ws, w2s)


def kernel(position_ids, hidden_states, norm1_w, norm1_b, norm2_w, norm2_b,
           Wqkv, Wout, Wrouter, ws, w2s):
    inv = 1.0 / (THETA ** (jnp.arange(HALF, dtype=jnp.float32) / HALF))
    ang = position_ids.astype(jnp.float32)[:, None] * inv[None, :]
    cos = jnp.cos(ang)
    sin = jnp.sin(ang)

    q, k, v = _run_k1(hidden_states, norm1_w, norm1_b, Wqkv, cos, sin)
    attn = _run_k2(q, k, v)
    h, x2, comb = _run_k3(attn, Wout, hidden_states, norm2_w, norm2_b, Wrouter)
    return _run_k4(x2, comb, h, ws, w2s)
